# baseline (speedup 1.0000x reference)
import functools

import jax
import jax.numpy as jnp
from jax import lax
from jax.experimental import pallas as pl
from jax.experimental.pallas import tpu as pltpu
from jax.experimental.pallas import tpu_sc as plsc

_VOCAB = 1000000
_DIM = 64
_B = 4096
_L = 200
_TOTAL = _B * _L
_NC = 2
_NS = 16
_NW = _NC * _NS
_PER_W = _TOTAL // _NW
_CHUNK = 1024
_NCHUNK = _PER_W // _CHUNK


def _gather_kernel(ids_hbm, table_hbm, out_hbm, idx_v, rows_v, sem):
    wid = lax.axis_index("s") * _NC + lax.axis_index("c")
    base = wid * _PER_W

    def step(c, carry):
        off = base + c * _CHUNK
        pltpu.sync_copy(ids_hbm.at[pl.ds(off, _CHUNK)], idx_v)
        pltpu.async_copy(table_hbm.at[idx_v], rows_v, sem).wait()
        pltpu.sync_copy(rows_v, out_hbm.at[pl.ds(off, _CHUNK)])
        return carry

    lax.fori_loop(0, _NCHUNK, step, 0)


@jax.jit
def kernel(input_ids, table):
    ids_flat = input_ids.reshape(_TOTAL)
    mesh = plsc.VectorSubcoreMesh(
        core_axis_name="c", subcore_axis_name="s",
        num_cores=_NC, num_subcores=_NS,
    )
    out = pl.kernel(
        _gather_kernel,
        out_type=jax.ShapeDtypeStruct((_TOTAL, _DIM), jnp.float32),
        mesh=mesh,
        scratch_types=[
            pltpu.VMEM((_CHUNK,), jnp.int32),
            pltpu.VMEM((_CHUNK, _DIM), jnp.float32),
            pltpu.SemaphoreType.DMA,
        ],
        compiler_params=pltpu.CompilerParams(use_tc_tiling_on_sc=False),
    )(ids_flat, table)
    return out.reshape(_B, _L, _DIM)


# speedup vs baseline: 1.0166x; 1.0166x over previous
import jax
import jax.numpy as jnp
from jax import lax
from jax.experimental import pallas as pl
from jax.experimental.pallas import tpu as pltpu
from jax.experimental.pallas import tpu_sc as plsc

_VOCAB = 1000000
_DIM = 64
_B = 4096
_L = 200
_TOTAL = _B * _L
_NC = 2
_NS = 16
_NW = _NC * _NS
_PER_W = _TOTAL // _NW
_CHUNK = 400
_NCHUNK = _PER_W // _CHUNK
_NBUF = 4


def _gather_kernel(ids_hbm, table_hbm, out_hbm, *scratch):
    idx = scratch[0:_NBUF]
    rows = scratch[_NBUF:2 * _NBUF]
    gsem = scratch[2 * _NBUF:3 * _NBUF]
    wsem = scratch[3 * _NBUF:4 * _NBUF]

    wid = lax.axis_index("s") * _NC + lax.axis_index("c")
    base = wid * _PER_W

    for b in range(_NBUF):
        pltpu.sync_copy(ids_hbm.at[pl.ds(base + b * _CHUNK, _CHUNK)], idx[b])
        pltpu.async_copy(table_hbm.at[idx[b]], rows[b], gsem[b])

    def step(k, carry):
        c = k * _NBUF
        for b in range(_NBUF):
            off = base + (c + b) * _CHUNK
            pltpu.make_async_copy(table_hbm.at[idx[b]], rows[b], gsem[b]).wait()
            pltpu.async_copy(rows[b], out_hbm.at[pl.ds(off, _CHUNK)], wsem[b])
        for b in range(_NBUF):
            off = base + (c + _NBUF + b) * _CHUNK
            pltpu.make_async_copy(
                rows[b], out_hbm.at[pl.ds(off, _CHUNK)], wsem[b]).wait()
            pltpu.sync_copy(ids_hbm.at[pl.ds(off, _CHUNK)], idx[b])
            pltpu.async_copy(table_hbm.at[idx[b]], rows[b], gsem[b])
        return carry

    lax.fori_loop(0, (_NCHUNK - 2 * _NBUF) // _NBUF + 1, step, 0)

    for b in range(_NBUF):
        off = base + (_NCHUNK - _NBUF + b) * _CHUNK
        pltpu.make_async_copy(table_hbm.at[idx[b]], rows[b], gsem[b]).wait()
        pltpu.async_copy(rows[b], out_hbm.at[pl.ds(off, _CHUNK)], wsem[b])
    for b in range(_NBUF):
        off = base + (_NCHUNK - _NBUF + b) * _CHUNK
        pltpu.make_async_copy(
            rows[b], out_hbm.at[pl.ds(off, _CHUNK)], wsem[b]).wait()


@jax.jit
def kernel(input_ids, table):
    ids_flat = input_ids.reshape(_TOTAL)
    mesh = plsc.VectorSubcoreMesh(
        core_axis_name="c", subcore_axis_name="s",
        num_cores=_NC, num_subcores=_NS,
    )
    out = pl.kernel(
        _gather_kernel,
        out_type=jax.ShapeDtypeStruct((_TOTAL, _DIM), jnp.float32),
        mesh=mesh,
        scratch_types=(
            [pltpu.VMEM((_CHUNK,), jnp.int32) for _ in range(_NBUF)]
            + [pltpu.VMEM((_CHUNK, _DIM), jnp.float32) for _ in range(_NBUF)]
            + [pltpu.SemaphoreType.DMA for _ in range(2 * _NBUF)]
        ),
        compiler_params=pltpu.CompilerParams(use_tc_tiling_on_sc=False),
    )(ids_flat, table)
    return out.reshape(_B, _L, _DIM)
